# baseline (speedup 1.0000x reference)
import functools

import jax
import jax.numpy as jnp
from jax import lax
from jax.experimental import pallas as pl
from jax.experimental.pallas import tpu as pltpu
from jax.experimental.pallas import tpu_sc as plsc

E = 320000
N_ATOMS = 10000
MAX_NB = 32
ATOM_FDIM = 128
BOND_FDIM = 144
H = 256
DEPTH = 6

NC, NS, L = 2, 16, 16
NW = NC * NS
N_PAD = 10240
ATOMS_W = N_PAD // NW
CH = 4
EDGES_W = E // NW
CE = 40

_mesh = plsc.VectorSubcoreMesh(
    core_axis_name="c", subcore_axis_name="s", num_cores=NC, num_subcores=NS)

f32 = jnp.float32


NCHUNK_S = ATOMS_W // CH
NCHUNK_D = EDGES_W // CE


def _segsum_body(msg_hbm, idx_hbm, out_hbm, idx_v, rows_v, acc_v, sg, so):
    wid = lax.axis_index("s") * NC + lax.axis_index("c")
    base_atom = wid * ATOMS_W

    def issue_gather(ci, b):
        for a in range(CH):
            pltpu.async_copy(
                msg_hbm.at[idx_v.at[pl.ds((ci * CH + a) * MAX_NB, MAX_NB)]],
                rows_v[b].at[pl.ds(a * MAX_NB, MAX_NB)], sg[b])

    def wait_gather(ci, b):
        for a in range(CH):
            pltpu.make_async_copy(
                msg_hbm.at[idx_v.at[pl.ds((ci * CH + a) * MAX_NB, MAX_NB)]],
                rows_v[b].at[pl.ds(a * MAX_NB, MAX_NB)], sg[b]).wait()

    pltpu.sync_copy(
        idx_hbm.at[pl.ds(base_atom * MAX_NB, ATOMS_W * MAX_NB)], idx_v)
    issue_gather(0, 0)

    def pair(cj, _):
        for b in range(2):
            ci = cj * 2 + b
            nxt = ci + 1

            @pl.when(nxt < NCHUNK_S)
            def _():
                issue_gather(nxt, 1 - b)

            wait_gather(ci, b)

            @pl.when(ci >= 2)
            def _():
                pltpu.make_async_copy(
                    acc_v[b], out_hbm.at[pl.ds(base_atom + (ci - 2) * CH, CH)],
                    so[b]).wait()

            def atom(a, _):
                r0 = a * MAX_NB
                for j in range(H // L):
                    vecs = [rows_v[b][r0 + r, pl.ds(j * L, L)]
                            for r in range(MAX_NB)]
                    while len(vecs) > 1:
                        vecs = [vecs[i] + vecs[i + 1]
                                for i in range(0, len(vecs), 2)]
                    acc_v[b][a, pl.ds(j * L, L)] = vecs[0]
                return 0

            lax.fori_loop(0, CH, atom, 0)
            pltpu.async_copy(
                acc_v[b], out_hbm.at[pl.ds(base_atom + ci * CH, CH)], so[b])
        return 0

    lax.fori_loop(0, NCHUNK_S // 2, pair, 0)
    for b in range(2):
        ci = NCHUNK_S - 2 + b
        pltpu.make_async_copy(
            acc_v[b], out_hbm.at[pl.ds(base_atom + ci * CH, CH)], so[b]).wait()


@functools.partial(
    pl.kernel,
    out_type=jax.ShapeDtypeStruct((N_PAD, H), f32),
    mesh=_mesh,
    scratch_types=[
        pltpu.VMEM((ATOMS_W * MAX_NB,), jnp.int32),
        [pltpu.VMEM((CH * MAX_NB, H), f32) for _ in range(2)],
        [pltpu.VMEM((CH, H), f32) for _ in range(2)],
        [pltpu.SemaphoreType.DMA for _ in range(2)],
        [pltpu.SemaphoreType.DMA for _ in range(2)],
    ],
)
def _segsum(msg_hbm, idx_hbm, out_hbm, idx_v, rows_v, acc_v, sg, so):
    _segsum_body(msg_hbm, idx_hbm, out_hbm, idx_v, rows_v, acc_v, sg, so)


def _combine_body(amh_hbm, mh_hbm, p_hbm, b2a_hbm, b2revb_hbm, out_hbm,
                  ia_v, ib_v, ra_v, rb_v, rp_v, d_v, sg, so):
    wid = lax.axis_index("s") * NC + lax.axis_index("c")
    base = wid * EDGES_W

    SPLITS = ((0, 24), (24, 16))

    def issue_gathers(ci, b):
        for o, n in SPLITS:
            pltpu.async_copy(
                amh_hbm.at[ia_v.at[pl.ds(ci * CE + o, n)]],
                ra_v[b].at[pl.ds(o, n)], sg[b])
            pltpu.async_copy(
                mh_hbm.at[ib_v.at[pl.ds(ci * CE + o, n)]],
                rb_v[b].at[pl.ds(o, n)], sg[b])
        pltpu.async_copy(
            p_hbm.at[pl.ds(base + ci * CE, CE)], rp_v[b], sg[b])

    pltpu.sync_copy(b2a_hbm.at[pl.ds(base, EDGES_W)], ia_v)
    pltpu.sync_copy(b2revb_hbm.at[pl.ds(base, EDGES_W)], ib_v)
    issue_gathers(0, 0)

    def pair(cj, _):
        for b in range(2):
            ci = cj * 2 + b
            nxt = ci + 1

            @pl.when(nxt < NCHUNK_D)
            def _():
                issue_gathers(nxt, 1 - b)

            for o, n in SPLITS:
                pltpu.make_async_copy(
                    amh_hbm.at[ia_v.at[pl.ds(ci * CE + o, n)]],
                    ra_v[b].at[pl.ds(o, n)], sg[b]).wait()
                pltpu.make_async_copy(
                    mh_hbm.at[ib_v.at[pl.ds(ci * CE + o, n)]],
                    rb_v[b].at[pl.ds(o, n)], sg[b]).wait()
            pltpu.make_async_copy(
                p_hbm.at[pl.ds(base + ci * CE, CE)], rp_v[b], sg[b]).wait()

            @pl.when(ci >= 2)
            def _():
                pltpu.make_async_copy(
                    d_v[b], out_hbm.at[pl.ds(base + (ci - 2) * CE, CE)],
                    so[b]).wait()

            def row(r, _):
                for u in range(2):
                    rr = r * 2 + u
                    for j in range(H // L):
                        sl = pl.ds(j * L, L)
                        v = rp_v[b][rr, sl] + ra_v[b][rr, sl] - rb_v[b][rr, sl]
                        d_v[b][rr, sl] = jnp.maximum(v, 0.0)
                return 0

            lax.fori_loop(0, CE // 2, row, 0)
            pltpu.async_copy(
                d_v[b], out_hbm.at[pl.ds(base + ci * CE, CE)], so[b])
        return 0

    lax.fori_loop(0, NCHUNK_D // 2, pair, 0)
    for b in range(2):
        ci = NCHUNK_D - 2 + b
        pltpu.make_async_copy(
            d_v[b], out_hbm.at[pl.ds(base + ci * CE, CE)], so[b]).wait()


@functools.partial(
    pl.kernel,
    out_type=jax.ShapeDtypeStruct((E, H), f32),
    mesh=_mesh,
    scratch_types=[
        pltpu.VMEM((EDGES_W,), jnp.int32),
        pltpu.VMEM((EDGES_W,), jnp.int32),
        [pltpu.VMEM((CE, H), f32) for _ in range(2)],
        [pltpu.VMEM((CE, H), f32) for _ in range(2)],
        [pltpu.VMEM((CE, H), f32) for _ in range(2)],
        [pltpu.VMEM((CE, H), f32) for _ in range(2)],
        [pltpu.SemaphoreType.DMA for _ in range(2)],
        [pltpu.SemaphoreType.DMA for _ in range(2)],
    ],
)
def _combine(amh_hbm, mh_hbm, p_hbm, b2a_hbm, b2revb_hbm, out_hbm,
             ia_v, ib_v, ra_v, rb_v, rp_v, d_v, sg, so):
    _combine_body(amh_hbm, mh_hbm, p_hbm, b2a_hbm, b2revb_hbm, out_hbm,
                  ia_v, ib_v, ra_v, rb_v, rp_v, d_v, sg, so)


BE = 512


def _init_body(fb_ref, wi_ref, bi_ref, bh_ref, m0_ref, p_ref):
    x = jnp.dot(fb_ref[...], wi_ref[...], preferred_element_type=f32,
                precision=lax.Precision.DEFAULT)
    x = x + bi_ref[...]
    m0_ref[...] = jnp.maximum(x, 0.0)
    p_ref[...] = x + bh_ref[...]


def _tc_init(f_bonds, W_i, b_i2, b_h2):
    return pl.pallas_call(
        _init_body,
        grid=(E // BE,),
        in_specs=[
            pl.BlockSpec((BE, BOND_FDIM), lambda i: (i, 0)),
            pl.BlockSpec((BOND_FDIM, H), lambda i: (0, 0)),
            pl.BlockSpec((1, H), lambda i: (0, 0)),
            pl.BlockSpec((1, H), lambda i: (0, 0)),
        ],
        out_specs=[
            pl.BlockSpec((BE, H), lambda i: (i, 0)),
            pl.BlockSpec((BE, H), lambda i: (i, 0)),
        ],
        out_shape=[
            jax.ShapeDtypeStruct((E, H), f32),
            jax.ShapeDtypeStruct((E, H), f32),
        ],
        compiler_params=pltpu.CompilerParams(
            dimension_semantics=("arbitrary",)),
    )(f_bonds, W_i, b_i2, b_h2)


def _mh_body(m_ref, wh_ref, out_ref):
    out_ref[...] = jnp.dot(m_ref[...], wh_ref[...], preferred_element_type=f32,
                           precision=lax.Precision.DEFAULT)


def _tc_mh(message, W_h):
    return pl.pallas_call(
        _mh_body,
        grid=(E // BE,),
        in_specs=[
            pl.BlockSpec((BE, H), lambda i: (i, 0)),
            pl.BlockSpec((H, H), lambda i: (0, 0)),
        ],
        out_specs=pl.BlockSpec((BE, H), lambda i: (i, 0)),
        out_shape=jax.ShapeDtypeStruct((E, H), f32),
        compiler_params=pltpu.CompilerParams(
            dimension_semantics=("arbitrary",)),
    )(message, W_h)


BA = 1024


def _tc_amh(am, W_h):
    return pl.pallas_call(
        _mh_body,
        grid=(N_PAD // BA,),
        in_specs=[
            pl.BlockSpec((BA, H), lambda i: (i, 0)),
            pl.BlockSpec((H, H), lambda i: (0, 0)),
        ],
        out_specs=pl.BlockSpec((BA, H), lambda i: (i, 0)),
        out_shape=jax.ShapeDtypeStruct((N_PAD, H), f32),
        compiler_params=pltpu.CompilerParams(
            dimension_semantics=("arbitrary",)),
    )(am, W_h)


BN = 400


def _final_body(fa_ref, am_ref, wo1_ref, wo2_ref, bo_ref, out_ref):
    acc = jnp.dot(fa_ref[...], wo1_ref[...], preferred_element_type=f32,
                  precision=lax.Precision.DEFAULT)
    acc = acc + jnp.dot(am_ref[...], wo2_ref[...], preferred_element_type=f32,
                        precision=lax.Precision.DEFAULT)
    out_ref[...] = jnp.maximum(acc + bo_ref[...], 0.0)


def _tc_final(f_atoms, am, W_o1, W_o2, b_o2):
    return pl.pallas_call(
        _final_body,
        grid=(N_ATOMS // BN,),
        in_specs=[
            pl.BlockSpec((BN, ATOM_FDIM), lambda i: (i, 0)),
            pl.BlockSpec((BN, H), lambda i: (i, 0)),
            pl.BlockSpec((ATOM_FDIM, H), lambda i: (0, 0)),
            pl.BlockSpec((H, H), lambda i: (0, 0)),
            pl.BlockSpec((1, H), lambda i: (0, 0)),
        ],
        out_specs=pl.BlockSpec((BN, H), lambda i: (i, 0)),
        out_shape=jax.ShapeDtypeStruct((N_ATOMS, H), f32),
        compiler_params=pltpu.CompilerParams(
            dimension_semantics=("arbitrary",)),
    )(f_atoms, am, W_o1, W_o2, b_o2)


def kernel(f_atoms, f_bonds, a2b, b2a, b2revb, W_i, b_i, W_h, b_h, W_o, b_o):
    a2b = a2b.astype(jnp.int32)
    b2a = b2a.astype(jnp.int32)
    b2revb = b2revb.astype(jnp.int32)
    a2b_pad = jnp.zeros((N_PAD, MAX_NB), jnp.int32).at[:N_ATOMS].set(a2b)
    a2b_flat = a2b_pad.reshape(-1)

    b_i2 = b_i.reshape(1, H)
    b_h2 = b_h.reshape(1, H)
    b_o2 = b_o.reshape(1, H)
    W_o1 = W_o[:ATOM_FDIM]
    W_o2 = W_o[ATOM_FDIM:]

    message, P = _tc_init(f_bonds, W_i, b_i2, b_h2)
    for _ in range(DEPTH - 1):
        am = _segsum(message, a2b_flat)
        mh = _tc_mh(message, W_h)
        amh = _tc_amh(am, W_h)
        message = _combine(amh, mh, P, b2a, b2revb)
    am5 = _segsum(message, a2b_flat)
    return _tc_final(f_atoms, am5[:N_ATOMS], W_o1, W_o2, b_o2)


# speedup vs baseline: 1.2228x; 1.2228x over previous
import functools

import jax
import jax.numpy as jnp
from jax import lax
from jax.experimental import pallas as pl
from jax.experimental.pallas import tpu as pltpu
from jax.experimental.pallas import tpu_sc as plsc

E = 320000
N_ATOMS = 10000
MAX_NB = 32
ATOM_FDIM = 128
BOND_FDIM = 144
H = 256
DEPTH = 6

NC, NS, L = 2, 16, 16
NW = NC * NS
N_PAD = 10240
ATOMS_W = N_PAD // NW
CH = 4
EDGES_W = E // NW
CE = 40

_mesh = plsc.VectorSubcoreMesh(
    core_axis_name="c", subcore_axis_name="s", num_cores=NC, num_subcores=NS)

f32 = jnp.float32


NCHUNK_S = ATOMS_W // CH
NCHUNK_D = EDGES_W // CE


def _segsum_body(msg_hbm, idx_hbm, out_hbm, idx_v, rows_v, acc_v, sg, so):
    wid = lax.axis_index("s") * NC + lax.axis_index("c")
    base_atom = wid * ATOMS_W

    def issue_gather(ci, b):
        for a in range(CH):
            pltpu.async_copy(
                msg_hbm.at[idx_v.at[pl.ds((ci * CH + a) * MAX_NB, MAX_NB)]],
                rows_v[b].at[pl.ds(a * MAX_NB, MAX_NB)], sg[b])

    def wait_gather(ci, b):
        for a in range(CH):
            pltpu.make_async_copy(
                msg_hbm.at[idx_v.at[pl.ds((ci * CH + a) * MAX_NB, MAX_NB)]],
                rows_v[b].at[pl.ds(a * MAX_NB, MAX_NB)], sg[b]).wait()

    pltpu.sync_copy(
        idx_hbm.at[pl.ds(base_atom * MAX_NB, ATOMS_W * MAX_NB)], idx_v)
    issue_gather(0, 0)

    def pair(cj, _):
        for b in range(2):
            ci = cj * 2 + b
            nxt = ci + 1

            @pl.when(nxt < NCHUNK_S)
            def _():
                issue_gather(nxt, 1 - b)

            wait_gather(ci, b)

            @pl.when(ci >= 2)
            def _():
                pltpu.make_async_copy(
                    acc_v[b], out_hbm.at[pl.ds(base_atom + (ci - 2) * CH, CH)],
                    so[b]).wait()

            def atom(a, _):
                r0 = a * MAX_NB
                for j in range(H // L):
                    vecs = [rows_v[b][r0 + r, pl.ds(j * L, L)]
                            for r in range(MAX_NB)]
                    while len(vecs) > 1:
                        vecs = [vecs[i] + vecs[i + 1]
                                for i in range(0, len(vecs), 2)]
                    acc_v[b][a, pl.ds(j * L, L)] = vecs[0]
                return 0

            lax.fori_loop(0, CH, atom, 0)
            pltpu.async_copy(
                acc_v[b], out_hbm.at[pl.ds(base_atom + ci * CH, CH)], so[b])
        return 0

    lax.fori_loop(0, NCHUNK_S // 2, pair, 0)
    for b in range(2):
        ci = NCHUNK_S - 2 + b
        pltpu.make_async_copy(
            acc_v[b], out_hbm.at[pl.ds(base_atom + ci * CH, CH)], so[b]).wait()


@functools.partial(
    pl.kernel,
    out_type=jax.ShapeDtypeStruct((N_PAD, H), f32),
    mesh=_mesh,
    scratch_types=[
        pltpu.VMEM((ATOMS_W * MAX_NB,), jnp.int32),
        [pltpu.VMEM((CH * MAX_NB, H), f32) for _ in range(2)],
        [pltpu.VMEM((CH, H), f32) for _ in range(2)],
        [pltpu.SemaphoreType.DMA for _ in range(2)],
        [pltpu.SemaphoreType.DMA for _ in range(2)],
    ],
)
def _segsum(msg_hbm, idx_hbm, out_hbm, idx_v, rows_v, acc_v, sg, so):
    _segsum_body(msg_hbm, idx_hbm, out_hbm, idx_v, rows_v, acc_v, sg, so)


def _combine_body(amh_hbm, mh_hbm, p_hbm, b2a_hbm, b2revb_hbm, out_hbm,
                  ia_v, ib_v, ra_v, rb_v, rp_v, d_v, sg, so):
    wid = lax.axis_index("s") * NC + lax.axis_index("c")
    base = wid * EDGES_W

    SPLITS = ((0, 24), (24, 16))

    def issue_gathers(ci, b):
        for o, n in SPLITS:
            pltpu.async_copy(
                amh_hbm.at[ia_v.at[pl.ds(ci * CE + o, n)]],
                ra_v[b].at[pl.ds(o, n)], sg[b])
            pltpu.async_copy(
                mh_hbm.at[ib_v.at[pl.ds(ci * CE + o, n)]],
                rb_v[b].at[pl.ds(o, n)], sg[b])
        pltpu.async_copy(
            p_hbm.at[pl.ds(base + ci * CE, CE)], rp_v[b], sg[b])

    pltpu.sync_copy(b2a_hbm.at[pl.ds(base, EDGES_W)], ia_v)
    pltpu.sync_copy(b2revb_hbm.at[pl.ds(base, EDGES_W)], ib_v)
    issue_gathers(0, 0)

    def pair(cj, _):
        for b in range(2):
            ci = cj * 2 + b
            nxt = ci + 1

            @pl.when(nxt < NCHUNK_D)
            def _():
                issue_gathers(nxt, 1 - b)

            for o, n in SPLITS:
                pltpu.make_async_copy(
                    amh_hbm.at[ia_v.at[pl.ds(ci * CE + o, n)]],
                    ra_v[b].at[pl.ds(o, n)], sg[b]).wait()
                pltpu.make_async_copy(
                    mh_hbm.at[ib_v.at[pl.ds(ci * CE + o, n)]],
                    rb_v[b].at[pl.ds(o, n)], sg[b]).wait()
            pltpu.make_async_copy(
                p_hbm.at[pl.ds(base + ci * CE, CE)], rp_v[b], sg[b]).wait()

            @pl.when(ci >= 2)
            def _():
                pltpu.make_async_copy(
                    d_v[b], out_hbm.at[pl.ds(base + (ci - 2) * CE, CE)],
                    so[b]).wait()

            def row(r, _):
                for u in range(2):
                    rr = r * 2 + u
                    for j in range(H // L):
                        sl = pl.ds(j * L, L)
                        v = rp_v[b][rr, sl] + ra_v[b][rr, sl] - rb_v[b][rr, sl]
                        d_v[b][rr, sl] = jnp.maximum(v, 0.0)
                return 0

            lax.fori_loop(0, CE // 2, row, 0)
            pltpu.async_copy(
                d_v[b], out_hbm.at[pl.ds(base + ci * CE, CE)], so[b])
        return 0

    lax.fori_loop(0, NCHUNK_D // 2, pair, 0)
    for b in range(2):
        ci = NCHUNK_D - 2 + b
        pltpu.make_async_copy(
            d_v[b], out_hbm.at[pl.ds(base + ci * CE, CE)], so[b]).wait()


@functools.partial(
    pl.kernel,
    out_type=jax.ShapeDtypeStruct((E, H), f32),
    mesh=_mesh,
    scratch_types=[
        pltpu.VMEM((EDGES_W,), jnp.int32),
        pltpu.VMEM((EDGES_W,), jnp.int32),
        [pltpu.VMEM((CE, H), f32) for _ in range(2)],
        [pltpu.VMEM((CE, H), f32) for _ in range(2)],
        [pltpu.VMEM((CE, H), f32) for _ in range(2)],
        [pltpu.VMEM((CE, H), f32) for _ in range(2)],
        [pltpu.SemaphoreType.DMA for _ in range(2)],
        [pltpu.SemaphoreType.DMA for _ in range(2)],
    ],
)
def _combine(amh_hbm, mh_hbm, p_hbm, b2a_hbm, b2revb_hbm, out_hbm,
             ia_v, ib_v, ra_v, rb_v, rp_v, d_v, sg, so):
    _combine_body(amh_hbm, mh_hbm, p_hbm, b2a_hbm, b2revb_hbm, out_hbm,
                  ia_v, ib_v, ra_v, rb_v, rp_v, d_v, sg, so)


BE = 512


def _init_body(fb_ref, wi_ref, bi_ref, bh_ref, m0_ref, p_ref):
    x = jnp.dot(fb_ref[...], wi_ref[...], preferred_element_type=f32,
                precision=lax.Precision.DEFAULT)
    x = x + bi_ref[...]
    m0_ref[...] = jnp.maximum(x, 0.0)
    p_ref[...] = x + bh_ref[...]


def _tc_init(f_bonds, W_i, b_i2, b_h2):
    return pl.pallas_call(
        _init_body,
        grid=(E // BE,),
        in_specs=[
            pl.BlockSpec((BE, BOND_FDIM), lambda i: (i, 0)),
            pl.BlockSpec((BOND_FDIM, H), lambda i: (0, 0)),
            pl.BlockSpec((1, H), lambda i: (0, 0)),
            pl.BlockSpec((1, H), lambda i: (0, 0)),
        ],
        out_specs=[
            pl.BlockSpec((BE, H), lambda i: (i, 0)),
            pl.BlockSpec((BE, H), lambda i: (i, 0)),
        ],
        out_shape=[
            jax.ShapeDtypeStruct((E, H), f32),
            jax.ShapeDtypeStruct((E, H), f32),
        ],
        compiler_params=pltpu.CompilerParams(
            dimension_semantics=("arbitrary",)),
    )(f_bonds, W_i, b_i2, b_h2)


def _mh_body(m_ref, wh_ref, out_ref):
    out_ref[...] = jnp.dot(m_ref[...], wh_ref[...], preferred_element_type=f32,
                           precision=lax.Precision.DEFAULT)


def _tc_mh(message, W_h):
    return pl.pallas_call(
        _mh_body,
        grid=(E // BE,),
        in_specs=[
            pl.BlockSpec((BE, H), lambda i: (i, 0)),
            pl.BlockSpec((H, H), lambda i: (0, 0)),
        ],
        out_specs=pl.BlockSpec((BE, H), lambda i: (i, 0)),
        out_shape=jax.ShapeDtypeStruct((E, H), f32),
        compiler_params=pltpu.CompilerParams(
            dimension_semantics=("arbitrary",)),
    )(message, W_h)


BA = 1024


def _tc_amh(am, W_h):
    return pl.pallas_call(
        _mh_body,
        grid=(N_PAD // BA,),
        in_specs=[
            pl.BlockSpec((BA, H), lambda i: (i, 0)),
            pl.BlockSpec((H, H), lambda i: (0, 0)),
        ],
        out_specs=pl.BlockSpec((BA, H), lambda i: (i, 0)),
        out_shape=jax.ShapeDtypeStruct((N_PAD, H), f32),
        compiler_params=pltpu.CompilerParams(
            dimension_semantics=("arbitrary",)),
    )(am, W_h)


BN = 400


def _final_body(fa_ref, am_ref, wo1_ref, wo2_ref, bo_ref, out_ref):
    acc = jnp.dot(fa_ref[...], wo1_ref[...], preferred_element_type=f32,
                  precision=lax.Precision.DEFAULT)
    acc = acc + jnp.dot(am_ref[...], wo2_ref[...], preferred_element_type=f32,
                        precision=lax.Precision.DEFAULT)
    out_ref[...] = jnp.maximum(acc + bo_ref[...], 0.0)


def _tc_final(f_atoms, am, W_o1, W_o2, b_o2):
    return pl.pallas_call(
        _final_body,
        grid=(N_ATOMS // BN,),
        in_specs=[
            pl.BlockSpec((BN, ATOM_FDIM), lambda i: (i, 0)),
            pl.BlockSpec((BN, H), lambda i: (i, 0)),
            pl.BlockSpec((ATOM_FDIM, H), lambda i: (0, 0)),
            pl.BlockSpec((H, H), lambda i: (0, 0)),
            pl.BlockSpec((1, H), lambda i: (0, 0)),
        ],
        out_specs=pl.BlockSpec((BN, H), lambda i: (i, 0)),
        out_shape=jax.ShapeDtypeStruct((N_ATOMS, H), f32),
        compiler_params=pltpu.CompilerParams(
            dimension_semantics=("arbitrary",)),
    )(f_atoms, am, W_o1, W_o2, b_o2)


def kernel(f_atoms, f_bonds, a2b, b2a, b2revb, W_i, b_i, W_h, b_h, W_o, b_o):
    a2b = a2b.astype(jnp.int32)
    b2a = b2a.astype(jnp.int32)
    b2revb = b2revb.astype(jnp.int32)
    n_pad_rows = N_PAD - N_ATOMS
    pad_idx = (jnp.arange(n_pad_rows * MAX_NB, dtype=jnp.int32) * 9973) % E
    a2b_pad = jnp.concatenate(
        [a2b, pad_idx.reshape(n_pad_rows, MAX_NB)], axis=0)
    a2b_flat = a2b_pad.reshape(-1)

    b_i2 = b_i.reshape(1, H)
    b_h2 = b_h.reshape(1, H)
    b_o2 = b_o.reshape(1, H)
    W_o1 = W_o[:ATOM_FDIM]
    W_o2 = W_o[ATOM_FDIM:]

    message, P = _tc_init(f_bonds, W_i, b_i2, b_h2)
    for _ in range(DEPTH - 1):
        am = _segsum(message, a2b_flat)
        mh = _tc_mh(message, W_h)
        amh = _tc_amh(am, W_h)
        message = _combine(amh, mh, P, b2a, b2revb)
    am5 = _segsum(message, a2b_flat)
    return _tc_final(f_atoms, am5[:N_ATOMS], W_o1, W_o2, b_o2)
